# baseline (speedup 1.0000x reference)
import jax
import jax.numpy as jnp
from jax import lax
from jax.experimental import pallas as pl
from jax.experimental.pallas import tpu as pltpu
from jax.experimental.pallas import tpu_sc as plsc

_PRUNE = 5
_B, _T, _U, _V = 4, 72, 64, 1024
_ND = _T + _U + 1
_DROWS = 144
_NFLAT = 2 * _DROWS * 128
_NW = 32
_PER_W = _NFLAT // _NW
_CHUNK = 128
_NCHUNK = _PER_W // _CHUNK
_VREGS = _PER_W // 16


def _sc_gather_body(lp_hbm, tgt_hbm, out_hbm, idx_v, vals_v, bias_v, tg_v, sem):
    nc = plsc.get_sparse_core_info().num_cores
    wid = lax.axis_index("s") * nc + lax.axis_index("c")
    w16 = lax.rem(wid, 16)
    is_lab = jnp.where(wid >= 16, jnp.int32(1), jnp.int32(0))
    pltpu.sync_copy(tgt_hbm, tg_v)
    b16 = lax.iota(jnp.int32, 16)
    bc = jnp.minimum(b16, _B - 1)
    bok = b16 < _B
    ninf = jnp.full((16,), -jnp.inf, jnp.float32)
    zero = jnp.zeros((16,), jnp.float32)
    copies = []
    for cc in range(_NCHUNK):
        for k in range(8):
            i = cc * 8 + k
            d = w16 * (_PER_W // 128) + (i // 8)
            j = i % 8
            t = j + (lax.div(d - _PRUNE + 1024, 2) - 512)
            u = d - t
            v1 = (t > 0) & (u < _U)
            v2 = (u > 0) & (t < _T)
            ok = (
                (t >= 0) & (t <= _T) & (u >= 0) & (u <= _U) & (d < _ND)
                & (u >= t - _PRUNE) & (u <= t + _PRUNE) & (v1 | v2)
            )
            keep = ok & jnp.where(is_lab == 1, v2, v1)
            tcl = jnp.clip(jnp.where(is_lab == 1, t, t - 1), 0, _T - 1)
            ucl = jnp.clip(jnp.where(is_lab == 1, u - 1, u), 0, _U - 1)
            tgv = tg_v[pl.ds(ucl * 16, 16)] * is_lab
            idx = (
                (bc * _T + tcl) * (_U * _V)
                + lax.div(ucl, 8) * (8 * _V)
                + lax.div(tgv, 128) * 1024
                + lax.rem(ucl, 8) * 128
                + lax.rem(tgv, 128)
            )
            idx_v[pl.ds(i * 16, 16)] = idx
            kbias = jnp.where(keep, jnp.float32(0.0), jnp.float32(-jnp.inf))
            bias_v[pl.ds(i * 16, 16)] = jnp.where(bok, zero, ninf) + kbias
        copies.append(
            pltpu.async_copy(
                lp_hbm.at[idx_v.at[pl.ds(cc * _CHUNK, _CHUNK)]],
                vals_v.at[pl.ds(cc * _CHUNK, _CHUNK)],
                sem,
            )
        )
    for cc in range(_NCHUNK):
        copies[cc].wait()
        for k in range(8):
            sl = pl.ds((cc * 8 + k) * 16, 16)
            vals_v[sl] = vals_v[sl] + bias_v[sl]
    pltpu.sync_copy(vals_v, out_hbm.at[pl.ds(wid * _PER_W, _PER_W)])


@jax.jit
def _sc_gather(lp_flat, tgt_t):
    mesh = plsc.VectorSubcoreMesh(core_axis_name="c", subcore_axis_name="s")
    run = pl.kernel(
        _sc_gather_body,
        out_type=jax.ShapeDtypeStruct((_NFLAT,), jnp.float32),
        mesh=mesh,
        scratch_types=[
            pltpu.VMEM((_PER_W,), jnp.int32),
            pltpu.VMEM((_PER_W,), jnp.float32),
            pltpu.VMEM((_PER_W,), jnp.float32),
            pltpu.VMEM((_U * 16,), jnp.int32),
            pltpu.SemaphoreType.DMA,
        ],
    )
    return run(lp_flat, tgt_t)


def _lse(a, b):
    m = jnp.maximum(a, b)
    ms = jnp.where(m == -jnp.inf, jnp.float32(0.0), m)
    return ms + jnp.log(jnp.exp(a - ms) + jnp.exp(b - ms))


def _dp_body(tab_ref, tlen_ref, ulen_ref, out_ref):
    ninf = jnp.float32(-jnp.inf)
    lane = lax.broadcasted_iota(jnp.int32, (1, 128), 1)
    j = lax.shift_right_logical(lane, 4)
    b = lax.bitwise_and(lane, 15)
    dstar_v = jnp.zeros((1, 128), jnp.int32)
    jstar_v = jnp.full((1, 128), -1, jnp.int32)
    tlen_v = jnp.full((1, 128), -1, jnp.int32)
    ulen_v = jnp.full((1, 128), -1, jnp.int32)
    for k in range(_B):
        tl = tlen_ref[k]
        ul = ulen_ref[k]
        ds_k = tl + ul
        js_k = tl - lax.shift_right_arithmetic(ds_k - _PRUNE, 1)
        sel = b == k
        dstar_v = jnp.where(sel, ds_k, dstar_v)
        jstar_v = jnp.where(sel, js_k, jstar_v)
        tlen_v = jnp.where(sel, tl, tlen_v)
        ulen_v = jnp.where(sel, ul, ulen_v)
    pick = (j == jstar_v) & (b < _B)
    prev0 = jnp.where(j == 3, jnp.float32(0.0), ninf)
    cap0 = jnp.where((dstar_v == 0) & pick, prev0, ninf)

    def step(d, carry):
        prev, cap = carry
        brow = tab_ref[pl.ds(d * 128, 128)].reshape(1, 128)
        lrow = tab_ref[pl.ds((_DROWS + d) * 128, 128)].reshape(1, 128)
        odd = lax.rem(d, 2) == 1
        t_vec = j + lax.shift_right_arithmetic(d - _PRUNE, 1)
        u_vec = d - t_vec
        lenok = (t_vec <= tlen_v) & (u_vec <= ulen_v)
        shl = jnp.concatenate([prev[:, 16:], jnp.full((1, 16), ninf)], axis=1)
        shr = jnp.concatenate([jnp.full((1, 16), ninf), prev[:, :112]], axis=1)
        c1 = jnp.where(odd, prev, shr) + brow
        c2 = jnp.where(odd, shl, prev) + lrow
        cur = jnp.where(lenok, _lse(c1, c2), ninf)
        cap = jnp.where((dstar_v == d) & pick, cur, cap)
        return (cur, cap)

    _, cap = lax.fori_loop(1, _ND, step, (prev0, cap0))
    loss = jnp.sum(jnp.where(pick, -cap, jnp.float32(0.0))) / _B
    out_ref[...] = jnp.full((8, 128), loss)


@jax.jit
def _dp(tab, tl, ul):
    out = pl.pallas_call(
        _dp_body,
        in_specs=[
            pl.BlockSpec(memory_space=pltpu.MemorySpace.VMEM),
            pl.BlockSpec(memory_space=pltpu.MemorySpace.SMEM),
            pl.BlockSpec(memory_space=pltpu.MemorySpace.SMEM),
        ],
        out_shape=jax.ShapeDtypeStruct((8, 128), jnp.float32),
    )(tab, tl, ul)
    return out[0, 0]


def kernel(log_probs, targets, logit_lengths, target_lengths):
    lp_flat = (
        log_probs.reshape(_B, _T, _U // 8, 8, _V // 128, 128)
        .transpose(0, 1, 2, 4, 3, 5)
        .reshape(-1)
    )
    tgt_t = jnp.pad(targets.astype(jnp.int32).T, ((0, 0), (0, 16 - _B))).reshape(-1)
    tab = _sc_gather(lp_flat, tgt_t)
    return _dp(tab, logit_lengths.astype(jnp.int32), target_lengths.astype(jnp.int32))
